# baseline (speedup 1.0000x reference)
import functools

import jax
import jax.numpy as jnp
from jax import lax
from jax.experimental import pallas as pl
from jax.experimental.pallas import tpu as pltpu
from jax.experimental.pallas import tpu_sc as plsc

B = 16384
H = 50
D = 32
NC, NS = 2, 16
NW = NC * NS
PB = B // NW
IW = 128
ND = PB // IW
NB = 2
L = 16


@functools.cache
def _build():
    mesh = plsc.VectorSubcoreMesh(
        core_axis_name="c", subcore_axis_name="s",
        num_cores=NC, num_subcores=NS)

    @functools.partial(
        pl.kernel,
        out_type=jax.ShapeDtypeStruct((H, D, B), jnp.float32),
        mesh=mesh,
        compiler_params=pltpu.CompilerParams(
            use_tc_tiling_on_sc=False, needs_layout_passes=False),
        scratch_types=[
            pltpu.VMEM((H, PB), jnp.int32),
            pltpu.VMEM((NB, PB, D), jnp.float32),
            pltpu.VMEM((NB, D, PB), jnp.float32),
            pltpu.SemaphoreType.DMA,
            pltpu.SemaphoreType.DMA,
        ],
    )
    def k(idx_hbm, table_hbm, out_hbm, idx_v, gbuf, tbuf, sem_g, sem_out):
        wid = lax.axis_index("s") * NC + lax.axis_index("c")
        base = wid * PB
        pltpu.sync_copy(idx_hbm.at[:, pl.ds(base, PB)], idx_v)
        lanes = lax.iota(jnp.int32, L)

        def transpose_block(b):
            @pl.loop(0, D)
            def per_d(d):
                dvec = jnp.full((L,), d, jnp.int32)
                for j in range(PB // L):
                    rvec = lanes + (j * L)
                    v = plsc.load_gather(gbuf.at[b], [rvec, dvec])
                    plsc.store_scatter(tbuf.at[b], [dvec, rvec], v)

        @pl.loop(0, H // NB)
        def body(g):
            @pl.when(g > 0)
            def _():
                for b in range(NB):
                    pltpu.make_async_copy(
                        tbuf.at[b], out_hbm.at[0, :, pl.ds(0, PB)], sem_out
                    ).wait()

            descs = []
            for b in range(NB):
                h = g * NB + b
                for q in range(ND):
                    d = pltpu.async_copy(
                        table_hbm.at[idx_v.at[h, pl.ds(q * IW, IW)]],
                        gbuf.at[b, pl.ds(q * IW, IW)],
                        sem_g)
                    descs.append(d)
            for b in range(NB):
                for q in range(ND):
                    descs[b * ND + q].wait()
                transpose_block(b)
                h = g * NB + b
                pltpu.async_copy(
                    tbuf.at[b],
                    out_hbm.at[h, :, pl.ds(base, PB)],
                    sem_out)

        for b in range(NB):
            pltpu.make_async_copy(
                tbuf.at[b], out_hbm.at[0, :, pl.ds(0, PB)], sem_out
            ).wait()

    return k


def kernel(label_idx, embedding_center):
    idx_t = jnp.transpose(label_idx.astype(jnp.int32))
    out_t = _build()(idx_t, embedding_center)
    return jnp.transpose(out_t, (2, 0, 1))


# speedup vs baseline: 1.2179x; 1.2179x over previous
import functools

import jax
import jax.numpy as jnp
from jax import lax
from jax.experimental import pallas as pl
from jax.experimental.pallas import tpu as pltpu
from jax.experimental.pallas import tpu_sc as plsc

B = 16384
H = 50
D = 32
NC, NS = 2, 16
NW = NC * NS
PB = B // NW
IW = 128
ND = PB // IW
NB = 2
L = 16


@functools.cache
def _build():
    mesh = plsc.VectorSubcoreMesh(
        core_axis_name="c", subcore_axis_name="s",
        num_cores=NC, num_subcores=NS)

    @functools.partial(
        pl.kernel,
        out_type=jax.ShapeDtypeStruct((H, D, B), jnp.float32),
        mesh=mesh,
        compiler_params=pltpu.CompilerParams(
            use_tc_tiling_on_sc=False, needs_layout_passes=False),
        scratch_types=[
            pltpu.VMEM((H, PB), jnp.int32),
            pltpu.VMEM((NB, PB, D), jnp.float32),
            pltpu.VMEM((NB, D, PB), jnp.float32),
            pltpu.SemaphoreType.DMA,
            pltpu.SemaphoreType.DMA,
        ],
    )
    def k(idx_hbm, table_hbm, out_hbm, idx_v, gbuf, tbuf, sem_g, sem_out):
        wid = lax.axis_index("s") * NC + lax.axis_index("c")
        base = wid * PB
        pltpu.sync_copy(idx_hbm.at[:, pl.ds(base, PB)], idx_v)
        lanes_lo = lax.iota(jnp.int32, L)
        lanes_hi = lanes_lo + L

        def transpose_block(b):
            @plsc.parallel_loop(0, PB, unroll=4)
            def per_row(r):
                v0 = gbuf[b, r, pl.ds(0, L)]
                v1 = gbuf[b, r, pl.ds(L, L)]
                rvec = jnp.full((L,), r, jnp.int32)
                plsc.store_scatter(tbuf.at[b], [lanes_lo, rvec], v0)
                plsc.store_scatter(tbuf.at[b], [lanes_hi, rvec], v1)

        @pl.loop(0, H // NB)
        def body(g):
            @pl.when(g > 0)
            def _():
                for b in range(NB):
                    pltpu.make_async_copy(
                        tbuf.at[b], out_hbm.at[0, :, pl.ds(0, PB)], sem_out
                    ).wait()

            descs = []
            for b in range(NB):
                h = g * NB + b
                for q in range(ND):
                    d = pltpu.async_copy(
                        table_hbm.at[idx_v.at[h, pl.ds(q * IW, IW)]],
                        gbuf.at[b, pl.ds(q * IW, IW)],
                        sem_g)
                    descs.append(d)
            for b in range(NB):
                for q in range(ND):
                    descs[b * ND + q].wait()
                transpose_block(b)
                h = g * NB + b
                pltpu.async_copy(
                    tbuf.at[b],
                    out_hbm.at[h, :, pl.ds(base, PB)],
                    sem_out)

        for b in range(NB):
            pltpu.make_async_copy(
                tbuf.at[b], out_hbm.at[0, :, pl.ds(0, PB)], sem_out
            ).wait()

    return k


def kernel(label_idx, embedding_center):
    idx_t = jnp.transpose(label_idx.astype(jnp.int32))
    out_t = _build()(idx_t, embedding_center)
    return jnp.transpose(out_t, (2, 0, 1))


# speedup vs baseline: 1.2429x; 1.0205x over previous
import functools

import jax
import jax.numpy as jnp
from jax import lax
from jax.experimental import pallas as pl
from jax.experimental.pallas import tpu as pltpu
from jax.experimental.pallas import tpu_sc as plsc

B = 16384
H = 50
D = 32
NC, NS = 2, 16
NW = NC * NS
PB = B // NW
IW = 128
ND = PB // IW
NB = 2
L = 16


@functools.cache
def _build():
    mesh = plsc.VectorSubcoreMesh(
        core_axis_name="c", subcore_axis_name="s",
        num_cores=NC, num_subcores=NS)

    @functools.partial(
        pl.kernel,
        out_type=jax.ShapeDtypeStruct((H, D, B), jnp.float32),
        mesh=mesh,
        compiler_params=pltpu.CompilerParams(
            use_tc_tiling_on_sc=False, needs_layout_passes=False),
        scratch_types=[
            pltpu.VMEM((H, PB), jnp.int32),
            pltpu.VMEM((NB, PB, D), jnp.float32),
            pltpu.VMEM((NB, D, PB), jnp.float32),
            pltpu.SemaphoreType.DMA,
            pltpu.SemaphoreType.DMA,
        ],
    )
    def k(idx_hbm, table_hbm, out_hbm, idx_v, gbuf, tbuf, sem_g, sem_out):
        wid = lax.axis_index("s") * NC + lax.axis_index("c")
        base = wid * PB
        pltpu.sync_copy(idx_hbm.at[:, pl.ds(base, PB)], idx_v)
        lanes_lo = lax.iota(jnp.int32, L)
        lanes_hi = lanes_lo + L

        def transpose_block(b):
            @plsc.parallel_loop(0, PB, unroll=8,
                                carry=jnp.zeros((L,), jnp.int32))
            def per_row(r, rvec):
                v0 = gbuf[b, r, pl.ds(0, L)]
                v1 = gbuf[b, r, pl.ds(L, L)]
                plsc.store_scatter(tbuf.at[b], [lanes_lo, rvec], v0)
                plsc.store_scatter(tbuf.at[b], [lanes_hi, rvec], v1)
                return rvec + 1

        @pl.loop(0, H // NB)
        def body(g):
            @pl.when(g > 0)
            def _():
                for b in range(NB):
                    pltpu.make_async_copy(
                        tbuf.at[b], out_hbm.at[0, :, pl.ds(0, PB)], sem_out
                    ).wait()

            descs = []
            for b in range(NB):
                h = g * NB + b
                for q in range(ND):
                    d = pltpu.async_copy(
                        table_hbm.at[idx_v.at[h, pl.ds(q * IW, IW)]],
                        gbuf.at[b, pl.ds(q * IW, IW)],
                        sem_g)
                    descs.append(d)
            for b in range(NB):
                for q in range(ND):
                    descs[b * ND + q].wait()
                transpose_block(b)
                h = g * NB + b
                pltpu.async_copy(
                    tbuf.at[b],
                    out_hbm.at[h, :, pl.ds(base, PB)],
                    sem_out)

        for b in range(NB):
            pltpu.make_async_copy(
                tbuf.at[b], out_hbm.at[0, :, pl.ds(0, PB)], sem_out
            ).wait()

    return k


def kernel(label_idx, embedding_center):
    idx_t = jnp.transpose(label_idx.astype(jnp.int32))
    out_t = _build()(idx_t, embedding_center)
    return jnp.transpose(out_t, (2, 0, 1))


# speedup vs baseline: 1.8268x; 1.4698x over previous
import functools

import jax
import jax.numpy as jnp
from jax import lax
from jax.experimental import pallas as pl
from jax.experimental.pallas import tpu as pltpu
from jax.experimental.pallas import tpu_sc as plsc

B = 16384
H = 50
D = 32
NC, NS = 2, 16
NW = NC * NS
PB = B // NW
IW = 128
ND = PB // IW
NB = 2
PBP = PB + 1
L = 16


@functools.cache
def _build():
    mesh = plsc.VectorSubcoreMesh(
        core_axis_name="c", subcore_axis_name="s",
        num_cores=NC, num_subcores=NS)

    @functools.partial(
        pl.kernel,
        out_type=jax.ShapeDtypeStruct((H, D, B), jnp.float32),
        mesh=mesh,
        compiler_params=pltpu.CompilerParams(
            use_tc_tiling_on_sc=False, needs_layout_passes=False),
        scratch_types=[
            pltpu.VMEM((H, PB), jnp.int32),
            pltpu.VMEM((NB, PB, D), jnp.float32),
            pltpu.VMEM((NB, D, PBP), jnp.float32),
            pltpu.SemaphoreType.DMA,
            pltpu.SemaphoreType.DMA,
        ],
    )
    def k(idx_hbm, table_hbm, out_hbm, idx_v, gbuf, tbuf, sem_g, sem_out):
        wid = lax.axis_index("s") * NC + lax.axis_index("c")
        base = wid * PB
        pltpu.sync_copy(idx_hbm.at[:, pl.ds(base, PB)], idx_v)
        lanes_lo = lax.iota(jnp.int32, L)
        lanes_hi = lanes_lo + L

        def transpose_block(b):
            @plsc.parallel_loop(0, PB, unroll=8,
                                carry=jnp.zeros((L,), jnp.int32))
            def per_row(r, rvec):
                v0 = gbuf[b, r, pl.ds(0, L)]
                v1 = gbuf[b, r, pl.ds(L, L)]
                plsc.store_scatter(tbuf.at[b], [lanes_lo, rvec], v0)
                plsc.store_scatter(tbuf.at[b], [lanes_hi, rvec], v1)
                return rvec + 1

        @pl.loop(0, H // NB)
        def body(g):
            @pl.when(g > 0)
            def _():
                for b in range(NB):
                    pltpu.make_async_copy(
                        tbuf.at[b, :, pl.ds(0, PB)],
                        out_hbm.at[0, :, pl.ds(0, PB)], sem_out
                    ).wait()

            descs = []
            for b in range(NB):
                h = g * NB + b
                for q in range(ND):
                    d = pltpu.async_copy(
                        table_hbm.at[idx_v.at[h, pl.ds(q * IW, IW)]],
                        gbuf.at[b, pl.ds(q * IW, IW)],
                        sem_g)
                    descs.append(d)
            for b in range(NB):
                for q in range(ND):
                    descs[b * ND + q].wait()
                transpose_block(b)
                h = g * NB + b
                pltpu.async_copy(
                    tbuf.at[b, :, pl.ds(0, PB)],
                    out_hbm.at[h, :, pl.ds(base, PB)],
                    sem_out)

        for b in range(NB):
            pltpu.make_async_copy(
                tbuf.at[b, :, pl.ds(0, PB)],
                out_hbm.at[0, :, pl.ds(0, PB)], sem_out
            ).wait()

    return k


def kernel(label_idx, embedding_center):
    idx_t = jnp.transpose(label_idx.astype(jnp.int32))
    out_t = _build()(idx_t, embedding_center)
    return jnp.transpose(out_t, (2, 0, 1))
